# baseline (speedup 1.0000x reference)
import functools

import jax
import jax.numpy as jnp
from jax.experimental import pallas as pl
from jax.experimental.pallas import tpu as pltpu

B = 1
T = 2048
D = 1024
H = 16
KV = 4
DH = D // H
NREP = H // KV
E = 8
F = 2048
L = 4

TT_PROJ = 512
TQ = 512
NPAIR = H // 2
TT_MOE = 512

BF = jnp.bfloat16
F32 = jnp.float32


def _ln(x, w, b):
    mu = jnp.mean(x, axis=-1, keepdims=True)
    xc = x - mu
    var = jnp.mean(xc * xc, axis=-1, keepdims=True)
    return xc * jax.lax.rsqrt(var + 1e-5) * w + b


def _dot(a, b):
    return jnp.dot(a, b, preferred_element_type=F32)


def _proj_body(x_ref, enc_ref, lnw_ref, lnb_ref, wq_ref, bq_ref,
               wk_ref, bk_ref, wv_ref, bv_ref,
               cwk_ref, cbk_ref, cwv_ref, cbv_ref,
               q_ref, k_ref, v_ref, ek_ref, ev_ref):
    xn = _ln(x_ref[...], lnw_ref[...], lnb_ref[...]).astype(BF)
    q_ref[...] = (_dot(xn, wq_ref[...]) + bq_ref[...]).astype(BF)
    enc = enc_ref[...].astype(BF)
    sel = (jax.lax.broadcasted_iota(jnp.int32, (xn.shape[0], DH), 1) == 0).astype(BF)
    for h in range(KV):
        k_ref[h] = (_dot(xn, wk_ref[h]) + bk_ref[h]).astype(BF)
        v_ref[h, :, :DH] = (_dot(xn, wv_ref[h]) + bv_ref[h]).astype(BF)
        v_ref[h, :, DH:] = sel
        ek_ref[h] = (_dot(enc, cwk_ref[h]) + cbk_ref[h]).astype(BF)
        ev_ref[h, :, :DH] = (_dot(enc, cwv_ref[h]) + cbv_ref[h]).astype(BF)
        ev_ref[h, :, DH:] = sel


def _proj(x, enc, lnw, lnb, wq, bq, wk, bk, wv, bv, cwk, cbk, cwv, cbv):
    nt = T // TT_PROJ
    full = lambda i: (0, 0)
    full3 = lambda i: (0, 0, 0)
    wspec = pl.BlockSpec((KV, D, DH), full3)
    bspec = pl.BlockSpec((KV, 1, DH), full3)
    kvspec = pl.BlockSpec((KV, TT_PROJ, DH), lambda i: (0, i, 0))
    vspec = pl.BlockSpec((KV, TT_PROJ, 2 * DH), lambda i: (0, i, 0))
    return pl.pallas_call(
        _proj_body,
        grid=(nt,),
        in_specs=[
            pl.BlockSpec((TT_PROJ, D), lambda i: (i, 0)),
            pl.BlockSpec((TT_PROJ, D), lambda i: (i, 0)),
            pl.BlockSpec((1, D), full),
            pl.BlockSpec((1, D), full),
            pl.BlockSpec((D, D), full),
            pl.BlockSpec((1, D), full),
            wspec, bspec, wspec, bspec,
            wspec, bspec, wspec, bspec,
        ],
        out_specs=[
            pl.BlockSpec((TT_PROJ, D), lambda i: (i, 0)),
            kvspec, vspec, kvspec, vspec,
        ],
        out_shape=[
            jax.ShapeDtypeStruct((T, D), BF),
            jax.ShapeDtypeStruct((KV, T, DH), BF),
            jax.ShapeDtypeStruct((KV, T, 2 * DH), BF),
            jax.ShapeDtypeStruct((KV, T, DH), BF),
            jax.ShapeDtypeStruct((KV, T, 2 * DH), BF),
        ],
    )(x, enc, lnw, lnb, wq, bq, wk, bk, wv, bv, cwk, cbk, cwv, cbv)


def _attn_block_body(next_is_q, q_ref, k_ref, v_ref, wo_ref, bo_ref, res_ref,
                     lnw_ref, lnb_ref, wn_ref, bn_ref,
                     hs_ref, nxt_ref, oacc_ref):
    p_id = pl.program_id(1)
    k = k_ref[0]
    v = v_ref[0]
    outs = []
    for j in range(2):
        q = q_ref[:, j * DH:(j + 1) * DH]
        s = jax.lax.dot_general(q, k, (((1,), (1,)), ((), ())),
                                preferred_element_type=F32)
        p = jnp.exp(s).astype(BF)
        ov = _dot(p, v)
        outs.append(ov[:, :DH] * (1.0 / ov[:, DH:DH + 1]))
    o2 = jnp.concatenate(outs, axis=-1).astype(BF)
    oacc_ref[:, pl.ds(p_id * 2 * DH, 2 * DH)] = o2

    @pl.when(p_id == NPAIR - 1)
    def _finish():
        hs = _dot(oacc_ref[...], wo_ref[...]) + bo_ref[...] + res_ref[...]
        hs_ref[...] = hs
        xn = _ln(hs, lnw_ref[...], lnb_ref[...]).astype(BF)
        if next_is_q:
            nxt_ref[...] = (_dot(xn, wn_ref[...]) + bn_ref[...]).astype(BF)
        else:
            nxt_ref[...] = xn


def _attn_block(q, k, v, wo, bo, res, lnw, lnb, wn, bn, next_is_q):
    nq = T // TQ
    full = lambda t, p: (0, 0)

    def tspec(t, p):
        return (t, 0)

    return pl.pallas_call(
        functools.partial(_attn_block_body, next_is_q),
        grid=(nq, NPAIR),
        in_specs=[
            pl.BlockSpec((TQ, 2 * DH), lambda t, p: (t, p)),
            pl.BlockSpec((1, T, DH), lambda t, p: (p // 2, 0, 0)),
            pl.BlockSpec((1, T, 2 * DH), lambda t, p: (p // 2, 0, 0)),
            pl.BlockSpec((D, D), full),
            pl.BlockSpec((1, D), full),
            pl.BlockSpec((TQ, D), tspec),
            pl.BlockSpec((1, D), full),
            pl.BlockSpec((1, D), full),
            pl.BlockSpec((D, D), full),
            pl.BlockSpec((1, D), full),
        ],
        out_specs=[
            pl.BlockSpec((TQ, D), tspec),
            pl.BlockSpec((TQ, D), tspec),
        ],
        out_shape=[
            jax.ShapeDtypeStruct((T, D), F32),
            jax.ShapeDtypeStruct((T, D), BF),
        ],
        scratch_shapes=[pltpu.VMEM((TQ, D), BF)],
    )(q, k, v, wo, bo, res, lnw, lnb, wn, bn)


def _active_cum(langs_ref):
    cum = []
    c = jnp.int32(0)
    for i in range(E):
        a = jnp.int32(0)
        for j in range(L):
            a = a | (langs_ref[j] == 4 + i).astype(jnp.int32)
        c = c + a
        cum.append(c)
    return cum


def _expert_for_slot(e, langs_ref):
    cum = _active_cum(langs_ref)
    n = cum[-1]
    e_c = jnp.minimum(e, jnp.maximum(n - 1, 0))
    p = jnp.int32(0)
    for i in range(E):
        p = p + (cum[i] <= e_c).astype(jnp.int32)
    return jnp.minimum(p, E - 1)


def _moe_body(langs_ref, x_ref, w1_ref, w3_ref, w2_ref, res_ref,
              out_ref, acc_ref):
    e = pl.program_id(0)
    t = pl.program_id(1)
    cum = _active_cum(langs_ref)
    n = cum[-1]

    @pl.when(e == 0)
    def _zero():
        acc_ref[pl.ds(t * TT_MOE, TT_MOE), :] = jnp.zeros((TT_MOE, D), F32)

    @pl.when(e < n)
    def _compute():
        x = x_ref[...]
        h1 = _dot(x, w1_ref[0])
        h3 = _dot(x, w3_ref[0])
        g = 0.5 * h1 * (1.0 + jax.lax.erf(h1 * (2.0 ** -0.5)))
        h = (g * h3).astype(BF)
        acc_ref[pl.ds(t * TT_MOE, TT_MOE), :] += _dot(h, w2_ref[0])

    @pl.when(e == E - 1)
    def _final():
        denom = jnp.int32(0)
        for j in range(L):
            denom = denom + (langs_ref[j] > 3).astype(jnp.int32)
        rw = jnp.where(denom > 0, 1.0 / jnp.maximum(denom, 1).astype(F32), 1.0)
        out_ref[...] = res_ref[...] + rw * acc_ref[pl.ds(t * TT_MOE, TT_MOE), :]


def _moe(x, langs, w1, w3, w2, res):
    nt = T // TT_MOE
    grid = (E, nt)

    def w13_idx(e, t, langs_ref):
        return (_expert_for_slot(e, langs_ref), 0, 0)

    def x_idx(e, t, langs_ref):
        cum = _active_cum(langs_ref)
        return (jnp.where(e < cum[-1], t, 0), 0)

    def res_idx(e, t, langs_ref):
        return (jnp.where(e == E - 1, t, 0), 0)

    grid_spec = pltpu.PrefetchScalarGridSpec(
        num_scalar_prefetch=1,
        grid=grid,
        in_specs=[
            pl.BlockSpec((TT_MOE, D), x_idx),
            pl.BlockSpec((1, D, F), w13_idx),
            pl.BlockSpec((1, D, F), w13_idx),
            pl.BlockSpec((1, F, D), w13_idx),
            pl.BlockSpec((TT_MOE, D), res_idx),
        ],
        out_specs=pl.BlockSpec((TT_MOE, D), res_idx),
        scratch_shapes=[pltpu.VMEM((T, D), F32)],
    )

    return pl.pallas_call(
        _moe_body,
        grid_spec=grid_spec,
        out_shape=jax.ShapeDtypeStruct((T, D), F32),
    )(langs, x, w1, w3, w2, res)


def kernel(hidden_states, encoder_hidden_states, attention_mask, langs,
           ln1_w, ln1_b, ln2_w, ln2_b, ln3_w, ln3_b,
           Wq, bq, Wk, bk, Wv, bv, Wo, bo,
           cWq, cbq, cWk, cbk, cWv, cbv, cWo, cbo,
           W1, W3, W2):
    hs = hidden_states.reshape(T, D)
    enc = encoder_hidden_states.reshape(T, D)
    lang = langs.reshape(L)
    r2 = lambda a: a.reshape(1, -1)
    bf = lambda a: a.astype(BF)
    sc = DH ** -0.5
    Wq_s, bq_s = bf(Wq * sc), (bq * sc).reshape(1, D)
    cWq_s, cbq_s = bf(cWq * sc), (cbq * sc).reshape(1, D)
    hm_w = lambda w: bf(w).reshape(D, KV, DH).transpose(1, 0, 2)
    hm_b = lambda b: b.reshape(KV, 1, DH)
    Wk_h, bk_h, Wv_h, bv_h = hm_w(Wk), hm_b(bk), hm_w(Wv), hm_b(bv)
    cWk_h, cbk_h, cWv_h, cbv_h = hm_w(cWk), hm_b(cbk), hm_w(cWv), hm_b(cbv)
    Wo_b, cWo_b = bf(Wo), bf(cWo)
    W1_b, W3_b, W2_b = bf(W1), bf(W3), bf(W2)

    q1, k1, v1, ek, ev = _proj(hs, enc, r2(ln1_w), r2(ln1_b), Wq_s, bq_s,
                               Wk_h, bk_h, Wv_h, bv_h, cWk_h, cbk_h, cWv_h, cbv_h)
    hs1, q2 = _attn_block(q1, k1, v1, Wo_b, r2(bo), hs, r2(ln2_w), r2(ln2_b),
                          cWq_s, cbq_s, next_is_q=True)
    hs2, xn3 = _attn_block(q2, ek, ev, cWo_b, r2(cbo), hs1, r2(ln3_w), r2(ln3_b),
                           cWo_b, r2(cbo), next_is_q=False)
    out = _moe(xn3, lang, W1_b, W3_b, W2_b, hs2)
    return out.reshape(B, T, D)


# speedup vs baseline: 1.1292x; 1.1292x over previous
import functools

import jax
import jax.numpy as jnp
from jax.experimental import pallas as pl
from jax.experimental.pallas import tpu as pltpu

B = 1
T = 2048
D = 1024
H = 16
KV = 4
DH = D // H
NREP = H // KV
E = 8
F = 2048
L = 4

TT_PROJ = 512
TQ = 512
NPAIR = H // 2
TT_MOE = 512

BF = jnp.bfloat16
F32 = jnp.float32


def _ln(x, w, b):
    mu = jnp.mean(x, axis=-1, keepdims=True)
    xc = x - mu
    var = jnp.mean(xc * xc, axis=-1, keepdims=True)
    return xc * jax.lax.rsqrt(var + 1e-5) * w + b


def _dot(a, b):
    return jnp.dot(a, b, preferred_element_type=F32)


def _proj_body(x_ref, enc_ref, lnw_ref, lnb_ref, wq_ref, bq_ref,
               wk_ref, bk_ref, wv_ref, bv_ref,
               cwk_ref, cbk_ref, cwv_ref, cbv_ref,
               q_ref, k_ref, v_ref, ek_ref, ev_ref):
    xn = _ln(x_ref[...], lnw_ref[...], lnb_ref[...]).astype(BF)
    q_ref[...] = (_dot(xn, wq_ref[...]) + bq_ref[...]).astype(BF)
    enc = enc_ref[...].astype(BF)
    sel = (jax.lax.broadcasted_iota(jnp.int32, (xn.shape[0], DH), 1) == 0).astype(BF)
    for h in range(KV):
        k_ref[h] = (_dot(xn, wk_ref[h]) + bk_ref[h]).astype(BF)
        v_ref[h, :, :DH] = (_dot(xn, wv_ref[h]) + bv_ref[h]).astype(BF)
        v_ref[h, :, DH:] = sel
        ek_ref[h] = (_dot(enc, cwk_ref[h]) + cbk_ref[h]).astype(BF)
        ev_ref[h, :, :DH] = (_dot(enc, cwv_ref[h]) + cbv_ref[h]).astype(BF)
        ev_ref[h, :, DH:] = sel


def _proj(x, enc, lnw, lnb, wq, bq, wk, bk, wv, bv, cwk, cbk, cwv, cbv):
    nt = T // TT_PROJ
    full = lambda i: (0, 0)
    full3 = lambda i: (0, 0, 0)
    wspec = pl.BlockSpec((KV, D, DH), full3)
    bspec = pl.BlockSpec((KV, 1, DH), full3)
    kvspec = pl.BlockSpec((KV, TT_PROJ, DH), lambda i: (0, i, 0))
    vspec = pl.BlockSpec((KV, TT_PROJ, 2 * DH), lambda i: (0, i, 0))
    return pl.pallas_call(
        _proj_body,
        grid=(nt,),
        in_specs=[
            pl.BlockSpec((TT_PROJ, D), lambda i: (i, 0)),
            pl.BlockSpec((TT_PROJ, D), lambda i: (i, 0)),
            pl.BlockSpec((1, D), full),
            pl.BlockSpec((1, D), full),
            pl.BlockSpec((D, D), full),
            pl.BlockSpec((1, D), full),
            wspec, bspec, wspec, bspec,
            wspec, bspec, wspec, bspec,
        ],
        out_specs=[
            pl.BlockSpec((TT_PROJ, D), lambda i: (i, 0)),
            kvspec, vspec, kvspec, vspec,
        ],
        out_shape=[
            jax.ShapeDtypeStruct((T, D), BF),
            jax.ShapeDtypeStruct((KV, T, DH), BF),
            jax.ShapeDtypeStruct((KV, T, 2 * DH), BF),
            jax.ShapeDtypeStruct((KV, T, DH), BF),
            jax.ShapeDtypeStruct((KV, T, 2 * DH), BF),
        ],
    )(x, enc, lnw, lnb, wq, bq, wk, bk, wv, bv, cwk, cbk, cwv, cbv)


def _attn_block_body(next_is_q, q_ref, k_ref, v_ref, wo_ref, bo_ref, res_ref,
                     lnw_ref, lnb_ref, wn_ref, bn_ref,
                     hs_ref, nxt_ref, oacc_ref):
    p_id = pl.program_id(1)
    k = k_ref[0]
    v = v_ref[0]
    outs = []
    for j in range(2):
        q = q_ref[:, j * DH:(j + 1) * DH]
        s = jax.lax.dot_general(q, k, (((1,), (1,)), ((), ())),
                                preferred_element_type=F32)
        p = jnp.exp(s).astype(BF)
        ov = _dot(p, v)
        outs.append(ov[:, :DH] * (1.0 / ov[:, DH:DH + 1]))
    o2 = jnp.concatenate(outs, axis=-1).astype(BF)
    oacc_ref[:, pl.ds(p_id * 2 * DH, 2 * DH)] = o2

    @pl.when(p_id == NPAIR - 1)
    def _finish():
        hs = _dot(oacc_ref[...], wo_ref[...]) + bo_ref[...] + res_ref[...]
        hs_ref[...] = hs
        xn = _ln(hs, lnw_ref[...], lnb_ref[...]).astype(BF)
        if next_is_q:
            nxt_ref[...] = (_dot(xn, wn_ref[...]) + bn_ref[...]).astype(BF)
        else:
            nxt_ref[...] = xn


def _attn_block(q, k, v, wo, bo, res, lnw, lnb, wn, bn, next_is_q):
    nq = T // TQ
    full = lambda t, p: (0, 0)

    def tspec(t, p):
        return (t, 0)

    return pl.pallas_call(
        functools.partial(_attn_block_body, next_is_q),
        grid=(nq, NPAIR),
        in_specs=[
            pl.BlockSpec((TQ, 2 * DH), lambda t, p: (t, p)),
            pl.BlockSpec((1, T, DH), lambda t, p: (p // 2, 0, 0)),
            pl.BlockSpec((1, T, 2 * DH), lambda t, p: (p // 2, 0, 0)),
            pl.BlockSpec((D, D), full),
            pl.BlockSpec((1, D), full),
            pl.BlockSpec((TQ, D), tspec),
            pl.BlockSpec((1, D), full),
            pl.BlockSpec((1, D), full),
            pl.BlockSpec((D, D), full),
            pl.BlockSpec((1, D), full),
        ],
        out_specs=[
            pl.BlockSpec((TQ, D), tspec),
            pl.BlockSpec((TQ, D), tspec),
        ],
        out_shape=[
            jax.ShapeDtypeStruct((T, D), F32),
            jax.ShapeDtypeStruct((T, D), BF),
        ],
        scratch_shapes=[pltpu.VMEM((TQ, D), BF)],
    )(q, k, v, wo, bo, res, lnw, lnb, wn, bn)


def _active_cum(langs_ref):
    cum = []
    c = jnp.int32(0)
    for i in range(E):
        a = jnp.int32(0)
        for j in range(L):
            a = a | (langs_ref[j] == 4 + i).astype(jnp.int32)
        c = c + a
        cum.append(c)
    return cum


def _expert_for_slot(e, langs_ref):
    cum = _active_cum(langs_ref)
    n = cum[-1]
    e_c = jnp.minimum(e, jnp.maximum(n - 1, 0))
    p = jnp.int32(0)
    for i in range(E):
        p = p + (cum[i] <= e_c).astype(jnp.int32)
    return jnp.minimum(p, E - 1)


FT = 2
FB = F // FT


def _moe_body(langs_ref, x_ref, w1_ref, w3_ref, w2_ref, res_ref,
              out_ref, acc_ref, w1s_ref, w3s_ref, w2s_ref):
    e = pl.program_id(0)
    f = pl.program_id(1)
    t = pl.program_id(2)
    cum = _active_cum(langs_ref)
    n = cum[-1]

    @pl.when((e == 0) & (f == 0))
    def _zero():
        acc_ref[pl.ds(t * TT_MOE, TT_MOE), :] = jnp.zeros((TT_MOE, D), F32)

    @pl.when((e < n) & (t == 0))
    def _cast():
        w1s_ref[...] = w1_ref[0].astype(BF)
        w3s_ref[...] = w3_ref[0].astype(BF)
        w2s_ref[...] = w2_ref[0].astype(BF)

    @pl.when(e < n)
    def _compute():
        x = x_ref[...]
        h1 = _dot(x, w1s_ref[...])
        h3 = _dot(x, w3s_ref[...])
        g = 0.5 * h1 * (1.0 + jax.lax.erf(h1 * (2.0 ** -0.5)))
        h = (g * h3).astype(BF)
        acc_ref[pl.ds(t * TT_MOE, TT_MOE), :] += _dot(h, w2s_ref[...])

    @pl.when((e == E - 1) & (f == FT - 1))
    def _final():
        denom = jnp.int32(0)
        for j in range(L):
            denom = denom + (langs_ref[j] > 3).astype(jnp.int32)
        rw = jnp.where(denom > 0, 1.0 / jnp.maximum(denom, 1).astype(F32), 1.0)
        out_ref[...] = res_ref[...] + rw * acc_ref[pl.ds(t * TT_MOE, TT_MOE), :]


def _moe(x, langs, w1, w3, w2, res):
    nt = T // TT_MOE
    grid = (E, FT, nt)

    def w13_idx(e, f, t, langs_ref):
        return (_expert_for_slot(e, langs_ref), 0, f)

    def w2_idx(e, f, t, langs_ref):
        return (_expert_for_slot(e, langs_ref), f, 0)

    def x_idx(e, f, t, langs_ref):
        cum = _active_cum(langs_ref)
        return (jnp.where(e < cum[-1], t, 0), 0)

    def res_idx(e, f, t, langs_ref):
        return (jnp.where((e == E - 1) & (f == FT - 1), t, 0), 0)

    grid_spec = pltpu.PrefetchScalarGridSpec(
        num_scalar_prefetch=1,
        grid=grid,
        in_specs=[
            pl.BlockSpec((TT_MOE, D), x_idx),
            pl.BlockSpec((1, D, FB), w13_idx),
            pl.BlockSpec((1, D, FB), w13_idx),
            pl.BlockSpec((1, FB, D), w2_idx),
            pl.BlockSpec((TT_MOE, D), res_idx),
        ],
        out_specs=pl.BlockSpec((TT_MOE, D), res_idx),
        scratch_shapes=[pltpu.VMEM((T, D), F32),
                        pltpu.VMEM((D, FB), BF),
                        pltpu.VMEM((D, FB), BF),
                        pltpu.VMEM((FB, D), BF)],
    )

    return pl.pallas_call(
        _moe_body,
        grid_spec=grid_spec,
        out_shape=jax.ShapeDtypeStruct((T, D), F32),
    )(langs, x, w1, w3, w2, res)


def kernel(hidden_states, encoder_hidden_states, attention_mask, langs,
           ln1_w, ln1_b, ln2_w, ln2_b, ln3_w, ln3_b,
           Wq, bq, Wk, bk, Wv, bv, Wo, bo,
           cWq, cbq, cWk, cbk, cWv, cbv, cWo, cbo,
           W1, W3, W2):
    hs = hidden_states.reshape(T, D)
    enc = encoder_hidden_states.reshape(T, D)
    lang = langs.reshape(L)
    r2 = lambda a: a.reshape(1, -1)
    bf = lambda a: a.astype(BF)
    sc = DH ** -0.5
    Wq_s, bq_s = bf(Wq * sc), (bq * sc).reshape(1, D)
    cWq_s, cbq_s = bf(cWq * sc), (cbq * sc).reshape(1, D)
    hm_w = lambda w: bf(w).reshape(D, KV, DH).transpose(1, 0, 2)
    hm_b = lambda b: b.reshape(KV, 1, DH)
    Wk_h, bk_h, Wv_h, bv_h = hm_w(Wk), hm_b(bk), hm_w(Wv), hm_b(bv)
    cWk_h, cbk_h, cWv_h, cbv_h = hm_w(cWk), hm_b(cbk), hm_w(cWv), hm_b(cbv)
    Wo_b, cWo_b = bf(Wo), bf(cWo)

    q1, k1, v1, ek, ev = _proj(hs, enc, r2(ln1_w), r2(ln1_b), Wq_s, bq_s,
                               Wk_h, bk_h, Wv_h, bv_h, cWk_h, cbk_h, cWv_h, cbv_h)
    hs1, q2 = _attn_block(q1, k1, v1, Wo_b, r2(bo), hs, r2(ln2_w), r2(ln2_b),
                          cWq_s, cbq_s, next_is_q=True)
    hs2, xn3 = _attn_block(q2, ek, ev, cWo_b, r2(cbo), hs1, r2(ln3_w), r2(ln3_b),
                           cWo_b, r2(cbo), next_is_q=False)
    out = _moe(xn3, lang, W1, W3, W2, hs2)
    return out.reshape(B, T, D)
